# Initial kernel scaffold; baseline (speedup 1.0000x reference)
import jax
import jax.numpy as jnp
from jax.experimental import pallas as pl


def kernel(user_table, item_table, u_w0, i_w0, u_w1, i_w1, u_cat_w, i_cat_w, vals0, vals1, rows0, cols0, rows1, cols1):
    raise NotImplementedError("write your pallas kernel here")


# speedup vs baseline: 10.7792x; 10.7792x over previous
import functools

import jax
import jax.numpy as jnp
from jax import lax
from jax.experimental import pallas as pl
from jax.experimental.pallas import tpu as pltpu
from jax.experimental.pallas import tpu_sc as plsc

N = 50000
D = 32
E = 800000

NTILE = 16
CHUNK = 128
TILE_CH = 392
EP = NTILE * TILE_CH * CHUNK
CH_TOT = EP // CHUNK
SEG = 28
NSEG = TILE_CH // SEG
NB = 2
NGRP = SEG // NB
NPAD = 50048
ZROW = 200


def _spmm_body(tab, meta, out, acc, mbuf, rbuf, zbuf, sem_g, sem_s):
    cid = lax.axis_index("c")
    sid = lax.axis_index("s")
    zv = jnp.zeros((16,), jnp.float32)

    @pl.loop(0, ZROW)
    def _zinit(i):
        zbuf[i, pl.ds(0, 16)] = zv
        zbuf[i, pl.ds(16, 16)] = zv

    for b in range(2):
        zb = sid * 3128
        for z in range(15):
            pltpu.sync_copy(zbuf, acc.at[pl.ds(zb + z * ZROW, ZROW)])
        pltpu.sync_copy(zbuf.at[pl.ds(0, 128)],
                        acc.at[pl.ds(zb + 3000, 128)])
        plsc.subcore_barrier()

        base_ch = sid * TILE_CH

        @pl.loop(0, NSEG)
        def _seg(h):
            seg0 = base_ch + h * SEG
            pltpu.sync_copy(meta.at[b, cid, pl.ds(seg0, SEG)], mbuf)

            @pl.loop(0, NGRP)
            def _grp(g):
                j0 = g * NB
                dg = [pltpu.async_copy(tab.at[mbuf.at[j0 + s, 0]],
                                       rbuf.at[s], sem_g)
                      for s in range(NB)]
                for d in dg:
                    d.wait()
                for s in range(NB):
                    @pl.loop(0, 8)
                    def _scale(gg, s=s):
                        vi = mbuf[j0 + s, 2, pl.ds(gg * 16, 16)]
                        vv = lax.bitcast_convert_type(vi, jnp.float32)
                        for e in range(16):
                            ei = gg * 16 + e
                            bv = lax.broadcast_in_dim(vv[e], (16,), ())
                            rbuf[s, ei, pl.ds(0, 16)] = (
                                rbuf[s, ei, pl.ds(0, 16)] * bv)
                            rbuf[s, ei, pl.ds(16, 16)] = (
                                rbuf[s, ei, pl.ds(16, 16)] * bv)
                ds_ = [pltpu.async_copy(rbuf.at[s],
                                        acc.at[mbuf.at[j0 + s, 1]],
                                        sem_s, add=True)
                       for s in range(NB)]
                for d in ds_:
                    d.wait()

        plsc.subcore_barrier()
        fb = sid * 3128

        @pl.when(sid < 15)
        def _flush_a(b=b):
            pltpu.sync_copy(acc.at[pl.ds(fb, 3128)],
                            out.at[cid, b, pl.ds(fb, 3128)])

        @pl.when(sid == 15)
        def _flush_b(b=b):
            pltpu.sync_copy(acc.at[pl.ds(fb, 3080)],
                            out.at[cid, b, pl.ds(fb, 3080)])

        plsc.subcore_barrier()


@jax.jit
def _spmm(tab, meta):
    mesh = plsc.VectorSubcoreMesh(core_axis_name="c", subcore_axis_name="s")
    return pl.kernel(
        _spmm_body,
        out_type=jax.ShapeDtypeStruct((2, 2, N, D), jnp.float32),
        mesh=mesh,
        scratch_types=[
            pltpu.VMEM_SHARED((NPAD, D), jnp.float32),
            pltpu.VMEM((SEG, 3, CHUNK), jnp.int32),
            pltpu.VMEM((NB, CHUNK, D), jnp.float32),
            pltpu.VMEM((ZROW, D), jnp.float32),
            pltpu.SemaphoreType.DMA,
            pltpu.SemaphoreType.DMA,
        ],
        compiler_params=pltpu.CompilerParams(use_tc_tiling_on_sc=False),
    )(tab, meta)


BLK = 2000


def _mid_body(side, agg_ref, w_ref, mo_ref, po_ref):
    wv = w_ref[...]
    a0 = agg_ref[0, 0]
    a1 = agg_ref[0, 1]
    m0 = jnp.dot(a0, wv, preferred_element_type=jnp.float32)
    m1 = jnp.dot(a1, wv, preferred_element_type=jnp.float32)
    po_ref[0] = jax.nn.sigmoid(m0)
    po_ref[1] = jax.nn.sigmoid(m1)
    mo_ref[...] = jax.nn.sigmoid((m0 + m1) * 0.5)


def _dense_mid(outcat, w, side):
    return pl.pallas_call(
        functools.partial(_mid_body, side),
        grid=(N // BLK,),
        in_specs=[
            pl.BlockSpec((1, 2, BLK, D), lambda i: (side, 0, i, 0)),
            pl.BlockSpec((D, D), lambda i: (0, 0)),
        ],
        out_specs=[
            pl.BlockSpec((BLK, D), lambda i: (i, 0)),
            pl.BlockSpec((2, BLK, D), lambda i: (0, i, 0)),
        ],
        out_shape=(
            jax.ShapeDtypeStruct((N, D), jnp.float32),
            jax.ShapeDtypeStruct((2, N, D), jnp.float32),
        ),
    )(outcat, w)


def _fin_body(side, agg_ref, w_ref, e1_ref, p1_ref, wc_ref, emb_ref, embs_ref):
    wv = w_ref[...]
    wc0 = wc_ref[pl.ds(0, D), :]
    wc1 = wc_ref[pl.ds(D, D), :]
    a0 = agg_ref[0, 0]
    a1 = agg_ref[0, 1]
    m0 = jnp.dot(a0, wv, preferred_element_type=jnp.float32)
    m1 = jnp.dot(a1, wv, preferred_element_type=jnp.float32)
    e2 = jax.nn.sigmoid((m0 + m1) * 0.5)
    p20 = jax.nn.sigmoid(m0)
    p21 = jax.nn.sigmoid(m1)
    dot = lambda x, y: jnp.dot(x, y, preferred_element_type=jnp.float32)
    emb_ref[...] = dot(e1_ref[...], wc0) + dot(e2, wc1)
    embs_ref[0] = dot(p1_ref[0], wc0) + dot(p20, wc1)
    embs_ref[1] = dot(p1_ref[1], wc0) + dot(p21, wc1)


def _dense_fin(outcat, w, e1, p1, wc, side):
    return pl.pallas_call(
        functools.partial(_fin_body, side),
        grid=(N // BLK,),
        in_specs=[
            pl.BlockSpec((1, 2, BLK, D), lambda i: (side, 0, i, 0)),
            pl.BlockSpec((D, D), lambda i: (0, 0)),
            pl.BlockSpec((BLK, D), lambda i: (i, 0)),
            pl.BlockSpec((2, BLK, D), lambda i: (0, i, 0)),
            pl.BlockSpec((2 * D, D), lambda i: (0, 0)),
        ],
        out_specs=[
            pl.BlockSpec((BLK, D), lambda i: (i, 0)),
            pl.BlockSpec((2, BLK, D), lambda i: (0, i, 0)),
        ],
        out_shape=(
            jax.ShapeDtypeStruct((N, D), jnp.float32),
            jax.ShapeDtypeStruct((2, N, D), jnp.float32),
        ),
    )(outcat, w, e1, p1, wc)


def _pad_i(x):
    return jnp.concatenate([x, jnp.zeros((EP - E,), x.dtype)])


def kernel(user_table, item_table, u_w0, i_w0, u_w1, i_w1, u_cat_w, i_cat_w,
           vals0, vals1, rows0, cols0, rows1, cols1):
    r0, c0 = _pad_i(rows0), _pad_i(cols0)
    r1, c1 = _pad_i(rows1), _pad_i(cols1)
    v0 = lax.bitcast_convert_type(_pad_i(vals0), jnp.int32)
    v1 = lax.bitcast_convert_type(_pad_i(vals1), jnp.int32)
    meta = jnp.stack([
        jnp.stack([jnp.stack([c0, r0, v0]), jnp.stack([r0 + N, c0, v0])]),
        jnp.stack([jnp.stack([c1, r1, v1]), jnp.stack([r1 + N, c1, v1])]),
    ]).reshape(2, 2, 3, CH_TOT, CHUNK).transpose(0, 1, 3, 2, 4)

    tab1 = jnp.concatenate([item_table, user_table], axis=0)
    out1 = _spmm(tab1, meta)
    ue1, ues1s = _dense_mid(out1, u_w0, side=0)
    ie1, ies1s = _dense_mid(out1, i_w0, side=1)

    tab2 = jnp.concatenate([ie1, ue1], axis=0)
    out2 = _spmm(tab2, meta)
    uemb, uembs = _dense_fin(out2, u_w1, ue1, ues1s, u_cat_w, side=0)
    iemb, iembs = _dense_fin(out2, i_w1, ie1, ies1s, i_cat_w, side=1)
    return (uemb, iemb, uembs, iembs)


# speedup vs baseline: 15.8047x; 1.4662x over previous
import functools

import jax
import jax.numpy as jnp
from jax import lax
from jax.experimental import pallas as pl
from jax.experimental.pallas import tpu as pltpu
from jax.experimental.pallas import tpu_sc as plsc

N = 50000
D = 32
E = 800000

NTILE = 16
CHUNK = 128
TILE_CH = 392
EP = NTILE * TILE_CH * CHUNK
CH_TOT = EP // CHUNK
SEG = 14
NSEG = TILE_CH // SEG
NB = 4
NPAD = 50048
ZROW = 96


def _spmm_body(tab, meta, out, acc, mbuf, rbuf, zbuf, sem_g, sem_s, sem_m):
    cid = lax.axis_index("c")
    sid = lax.axis_index("s")
    zv = jnp.zeros((16,), jnp.float32)

    @pl.loop(0, ZROW)
    def _zinit(i):
        zbuf[i, pl.ds(0, 16)] = zv
        zbuf[i, pl.ds(16, 16)] = zv

    def drain(sem, n):
        for _ in range(n):
            pltpu.make_async_copy(tab.at[pl.ds(0, CHUNK)],
                                  rbuf.at[0], sem).wait()

    for b in range(2):
        zb = sid * 3128
        dz = [pltpu.async_copy(zbuf, acc.at[pl.ds(zb + z * ZROW, ZROW)],
                               sem_g) for z in range(32)]
        dz.append(pltpu.async_copy(zbuf.at[pl.ds(0, 56)],
                                   acc.at[pl.ds(zb + 3072, 56)], sem_g))
        for d in dz:
            d.wait()
        plsc.subcore_barrier()

        base_ch = sid * TILE_CH
        pltpu.sync_copy(meta.at[b, cid, pl.ds(base_ch, SEG)], mbuf.at[0])

        @pl.loop(0, NSEG)
        def _seg(h):
            slot = h & 1
            seg0 = base_ch + h * SEG
            nseg0 = base_ch + jnp.minimum(h + 1, NSEG - 1) * SEG
            dm = pltpu.async_copy(meta.at[b, cid, pl.ds(nseg0, SEG)],
                                  mbuf.at[1 - slot], sem_m)
            pltpu.async_copy(tab.at[mbuf.at[slot, 0, 0]],
                             rbuf.at[(h * SEG) & 3], sem_g)
            pltpu.async_copy(tab.at[mbuf.at[slot, 1, 0]],
                             rbuf.at[(h * SEG + 1) & 3], sem_g)

            @pl.loop(0, SEG)
            def _chunk(j, slot=slot, seg0=seg0):
                jg = h * SEG + j
                s = jg & 3

                @pl.when(jg >= 2)
                def _():
                    drain(sem_s, 1)

                @pl.when(j + 2 < SEG)
                def _():
                    pltpu.async_copy(tab.at[mbuf.at[slot, j + 2, 0]],
                                     rbuf.at[(jg + 2) & 3], sem_g)

                drain(sem_g, 1)

                @pl.loop(0, 8)
                def _scale(gg):
                    vi = mbuf[slot, j, 2, pl.ds(gg * 16, 16)]
                    vv = lax.bitcast_convert_type(vi, jnp.float32)
                    for e in range(16):
                        ei = gg * 16 + e
                        bv = lax.broadcast_in_dim(vv[e], (16,), ())
                        rbuf[s, ei, pl.ds(0, 16)] = (
                            rbuf[s, ei, pl.ds(0, 16)] * bv)
                        rbuf[s, ei, pl.ds(16, 16)] = (
                            rbuf[s, ei, pl.ds(16, 16)] * bv)

                pltpu.async_copy(rbuf.at[s], acc.at[mbuf.at[slot, j, 1]],
                                 sem_s, add=True)

            dm.wait()

        drain(sem_s, 2)
        plsc.subcore_barrier()
        fb = sid * 3128

        @pl.when(sid < 15)
        def _flush_a(b=b):
            pltpu.sync_copy(acc.at[pl.ds(fb, 3128)],
                            out.at[cid, b, pl.ds(fb, 3128)])

        @pl.when(sid == 15)
        def _flush_b(b=b):
            pltpu.sync_copy(acc.at[pl.ds(fb, 3080)],
                            out.at[cid, b, pl.ds(fb, 3080)])

        plsc.subcore_barrier()


@jax.jit
def _spmm(tab, meta):
    mesh = plsc.VectorSubcoreMesh(core_axis_name="c", subcore_axis_name="s")
    return pl.kernel(
        _spmm_body,
        out_type=jax.ShapeDtypeStruct((2, 2, N, D), jnp.float32),
        mesh=mesh,
        scratch_types=[
            pltpu.VMEM_SHARED((NPAD, D), jnp.float32),
            pltpu.VMEM((2, SEG, 3, CHUNK), jnp.int32),
            pltpu.VMEM((NB, CHUNK, D), jnp.float32),
            pltpu.VMEM((ZROW, D), jnp.float32),
            pltpu.SemaphoreType.DMA,
            pltpu.SemaphoreType.DMA,
            pltpu.SemaphoreType.DMA,
        ],
        compiler_params=pltpu.CompilerParams(use_tc_tiling_on_sc=False),
    )(tab, meta)


BLK = 2000


def _mid_body(side, agg_ref, w_ref, mo_ref, po_ref):
    wv = w_ref[...]
    a0 = agg_ref[0, 0]
    a1 = agg_ref[0, 1]
    m0 = jnp.dot(a0, wv, preferred_element_type=jnp.float32)
    m1 = jnp.dot(a1, wv, preferred_element_type=jnp.float32)
    po_ref[0] = jax.nn.sigmoid(m0)
    po_ref[1] = jax.nn.sigmoid(m1)
    mo_ref[...] = jax.nn.sigmoid((m0 + m1) * 0.5)


def _dense_mid(outcat, w, side):
    return pl.pallas_call(
        functools.partial(_mid_body, side),
        grid=(N // BLK,),
        in_specs=[
            pl.BlockSpec((1, 2, BLK, D), lambda i: (side, 0, i, 0)),
            pl.BlockSpec((D, D), lambda i: (0, 0)),
        ],
        out_specs=[
            pl.BlockSpec((BLK, D), lambda i: (i, 0)),
            pl.BlockSpec((2, BLK, D), lambda i: (0, i, 0)),
        ],
        out_shape=(
            jax.ShapeDtypeStruct((N, D), jnp.float32),
            jax.ShapeDtypeStruct((2, N, D), jnp.float32),
        ),
    )(outcat, w)


def _fin_body(side, agg_ref, w_ref, e1_ref, p1_ref, wc_ref, emb_ref, embs_ref):
    wv = w_ref[...]
    wc0 = wc_ref[pl.ds(0, D), :]
    wc1 = wc_ref[pl.ds(D, D), :]
    a0 = agg_ref[0, 0]
    a1 = agg_ref[0, 1]
    m0 = jnp.dot(a0, wv, preferred_element_type=jnp.float32)
    m1 = jnp.dot(a1, wv, preferred_element_type=jnp.float32)
    e2 = jax.nn.sigmoid((m0 + m1) * 0.5)
    p20 = jax.nn.sigmoid(m0)
    p21 = jax.nn.sigmoid(m1)
    dot = lambda x, y: jnp.dot(x, y, preferred_element_type=jnp.float32)
    emb_ref[...] = dot(e1_ref[...], wc0) + dot(e2, wc1)
    embs_ref[0] = dot(p1_ref[0], wc0) + dot(p20, wc1)
    embs_ref[1] = dot(p1_ref[1], wc0) + dot(p21, wc1)


def _dense_fin(outcat, w, e1, p1, wc, side):
    return pl.pallas_call(
        functools.partial(_fin_body, side),
        grid=(N // BLK,),
        in_specs=[
            pl.BlockSpec((1, 2, BLK, D), lambda i: (side, 0, i, 0)),
            pl.BlockSpec((D, D), lambda i: (0, 0)),
            pl.BlockSpec((BLK, D), lambda i: (i, 0)),
            pl.BlockSpec((2, BLK, D), lambda i: (0, i, 0)),
            pl.BlockSpec((2 * D, D), lambda i: (0, 0)),
        ],
        out_specs=[
            pl.BlockSpec((BLK, D), lambda i: (i, 0)),
            pl.BlockSpec((2, BLK, D), lambda i: (0, i, 0)),
        ],
        out_shape=(
            jax.ShapeDtypeStruct((N, D), jnp.float32),
            jax.ShapeDtypeStruct((2, N, D), jnp.float32),
        ),
    )(outcat, w, e1, p1, wc)


def _pad_i(x):
    return jnp.concatenate([x, jnp.zeros((EP - E,), x.dtype)])


def kernel(user_table, item_table, u_w0, i_w0, u_w1, i_w1, u_cat_w, i_cat_w,
           vals0, vals1, rows0, cols0, rows1, cols1):
    r0, c0 = _pad_i(rows0), _pad_i(cols0)
    r1, c1 = _pad_i(rows1), _pad_i(cols1)
    v0 = lax.bitcast_convert_type(_pad_i(vals0), jnp.int32)
    v1 = lax.bitcast_convert_type(_pad_i(vals1), jnp.int32)
    meta = jnp.stack([
        jnp.stack([jnp.stack([c0, r0, v0]), jnp.stack([r0 + N, c0, v0])]),
        jnp.stack([jnp.stack([c1, r1, v1]), jnp.stack([r1 + N, c1, v1])]),
    ]).reshape(2, 2, 3, CH_TOT, CHUNK).transpose(0, 1, 3, 2, 4)

    tab1 = jnp.concatenate([item_table, user_table], axis=0)
    out1 = _spmm(tab1, meta)
    ue1, ues1s = _dense_mid(out1, u_w0, side=0)
    ie1, ies1s = _dense_mid(out1, i_w0, side=1)

    tab2 = jnp.concatenate([ie1, ue1], axis=0)
    out2 = _spmm(tab2, meta)
    uemb, uembs = _dense_fin(out2, u_w1, ue1, ues1s, u_cat_w, side=0)
    iemb, iembs = _dense_fin(out2, i_w1, ie1, ies1s, i_cat_w, side=1)
    return (uemb, iemb, uembs, iembs)
